# baseline (speedup 1.0000x reference)
import functools
import math

import jax
import jax.numpy as jnp
from jax import lax
from jax.experimental import pallas as pl
from jax.experimental.pallas import tpu as pltpu
from jax.experimental.pallas import tpu_sc as plsc

N = 10000
NG = 512
H = 128
F = 128
L = 6
G = 50
CUT = 10.0
K = 32
OUT = 4
P = 64

_BN = 256
_BE = _BN * K
_NP = 10240
_EP = _NP * K
_NW = 32
_PER_W = _EP // _NW
_CH = 128
_NCH = _PER_W // _CH


def _ssp(x):
    return jax.nn.softplus(x) - jnp.log(2.0)


def _build_graph_full(pos, batch):
    n = pos.shape[0]
    sq = jnp.sum(pos * pos, axis=-1)
    all_idx = jnp.arange(n)
    srcs, masks = [], []
    block = 1000
    for s in range(0, n, block):
        pb = pos[s:s + block]
        nb = pb.shape[0]
        rows = jnp.arange(s, s + nb)
        d2 = sq[s:s + nb, None] + sq[None, :] - 2.0 * (pb @ pos.T)
        valid = (batch[s:s + nb, None] == batch[None, :]) & (
            rows[:, None] != all_idx[None, :]) & (d2 <= CUT * CUT)
        d2m = jnp.where(valid, d2, jnp.inf)
        vals, idx = jax.lax.top_k(-d2m, K)
        srcs.append(idx.reshape(-1))
        masks.append(jnp.isfinite(vals).reshape(-1))
    return jnp.concatenate(srcs).reshape(N, K), jnp.concatenate(masks).reshape(N, K)


def _build_graph_windowed(pos, batch, seg_start, seg_len):
    slot = jnp.arange(P)
    gidx = seg_start[:, None] + slot[None, :]
    valid_slot = slot[None, :] < seg_len[:, None]
    gidx_c = jnp.where(valid_slot, gidx, N)
    posp = jnp.concatenate([pos, jnp.full((1, 3), 1e9, pos.dtype)], axis=0)
    gpos = posp[gidx_c]
    sq = jnp.sum(gpos * gpos, axis=-1)
    d2 = sq[:, :, None] + sq[:, None, :] - 2.0 * jnp.einsum(
        "gpc,gqc->gpq", gpos, gpos)
    eye = slot[:, None] == slot[None, :]
    valid = valid_slot[:, None, :] & (~eye)[None, :, :] & (d2 <= CUT * CUT)
    d2m = jnp.where(valid, d2, jnp.inf)
    vals, idx = jax.lax.top_k(-d2m.reshape(NG * P, P), K)
    mask_rows = jnp.isfinite(vals)
    src_rows = (seg_start[:, None, None] + idx.reshape(NG, P, K)).reshape(
        NG * P, K)
    src_rows = jnp.where(mask_rows, src_rows, 0)
    rows = batch * P + (jnp.arange(N) - seg_start[batch])
    return src_rows[rows], mask_rows[rows]


def _build_radius_graph(pos, batch):
    pos = jax.lax.stop_gradient(pos)
    batch = batch.astype(jnp.int32)
    gids = jnp.arange(NG, dtype=batch.dtype)
    seg_start = jnp.searchsorted(batch, gids, side="left").astype(jnp.int32)
    seg_end = jnp.searchsorted(batch, gids, side="right").astype(jnp.int32)
    seg_len = seg_end - seg_start
    overflow = jnp.max(seg_len) > P
    src, mask = jax.lax.cond(
        overflow,
        lambda: _build_graph_full(pos, batch),
        lambda: _build_graph_windowed(pos, batch, seg_start, seg_len),
    )
    return src.reshape(-1), mask.reshape(-1)


def _sc_gather_body(xs_hbm, idx_hbm, out_hbm, idx_v, buf, sem):
    c = lax.axis_index("c")
    s = lax.axis_index("s")
    wid = s * 2 + c
    pltpu.sync_copy(idx_hbm.at[wid], idx_v)
    base = wid * _PER_W

    def step(i, _):
        pltpu.async_copy(xs_hbm.at[idx_v.at[i]], buf, sem).wait()
        pltpu.sync_copy(buf, out_hbm.at[pl.ds(base + i * _CH, _CH)])
        return 0

    lax.fori_loop(0, _NCH, step, 0)


def _sc_gather(xs, src):
    mesh = plsc.VectorSubcoreMesh(core_axis_name="c", subcore_axis_name="s")
    idx3 = src.reshape(_NW, _NCH, _CH)
    f = pl.kernel(
        _sc_gather_body,
        out_type=jax.ShapeDtypeStruct((_EP, H), jnp.float32),
        mesh=mesh,
        scratch_types=[
            pltpu.VMEM((_NCH, _CH), jnp.int32),
            pltpu.VMEM((_CH, H), jnp.float32),
            pltpu.SemaphoreType.DMA,
        ],
    )
    return f(xs, idx3)


_COEFF = -0.5 / (CUT / (G - 1)) ** 2


def _layer_body(d_ref, c_ref, xg_ref, h_ref, w1_ref, b1_ref, w2_ref, b2_ref,
                cl2_ref, bcl2_ref, wlin_ref, blin_ref, cl1n_ref,
                hout_ref, xsout_ref):
    d = d_ref[...]
    offs = lax.broadcasted_iota(jnp.int32, (_BE, G), 1).astype(
        jnp.float32) * (CUT / (G - 1))
    ea = jnp.exp(_COEFF * (d - offs) ** 2)
    t = _ssp(jnp.dot(ea, w1_ref[...],
                     preferred_element_type=jnp.float32) + b1_ref[...])
    w = (jnp.dot(t, w2_ref[...],
                 preferred_element_type=jnp.float32) + b2_ref[...]) * c_ref[...]
    msg = w * xg_ref[...]
    agg = jnp.sum(msg.reshape(_BN, K, H), axis=1)
    conv = jnp.dot(agg, cl2_ref[...],
                   preferred_element_type=jnp.float32) + bcl2_ref[...]
    hn = h_ref[...] + jnp.dot(_ssp(conv), wlin_ref[...],
                              preferred_element_type=jnp.float32) + blin_ref[...]
    hout_ref[...] = hn
    xsout_ref[...] = jnp.dot(hn, cl1n_ref[...],
                             preferred_element_type=jnp.float32)


def _layer(d2d, c2d, xg, h, w1, b1, w2, b2, cl2, bcl2, wlin, blin, cl1n):
    nb = _NP // _BN
    full = lambda i: (0, 0)
    outs = pl.pallas_call(
        _layer_body,
        grid=(nb,),
        in_specs=[
            pl.BlockSpec((_BE, 1), lambda i: (i, 0)),
            pl.BlockSpec((_BE, 1), lambda i: (i, 0)),
            pl.BlockSpec((_BE, H), lambda i: (i, 0)),
            pl.BlockSpec((_BN, H), lambda i: (i, 0)),
            pl.BlockSpec((G, F), full),
            pl.BlockSpec((F,), lambda i: (0,)),
            pl.BlockSpec((F, F), full),
            pl.BlockSpec((F,), lambda i: (0,)),
            pl.BlockSpec((F, H), full),
            pl.BlockSpec((H,), lambda i: (0,)),
            pl.BlockSpec((H, H), full),
            pl.BlockSpec((H,), lambda i: (0,)),
            pl.BlockSpec((H, F), full),
        ],
        out_specs=[
            pl.BlockSpec((_BN, H), lambda i: (i, 0)),
            pl.BlockSpec((_BN, F), lambda i: (i, 0)),
        ],
        out_shape=[
            jax.ShapeDtypeStruct((_NP, H), jnp.float32),
            jax.ShapeDtypeStruct((_NP, F), jnp.float32),
        ],
    )(d2d, c2d, xg, h, w1, b1, w2, b2, cl2, bcl2, wlin, blin, cl1n)
    return outs


def _xs0_body(h_ref, w_ref, out_ref):
    out_ref[...] = jnp.dot(h_ref[...], w_ref[...],
                           preferred_element_type=jnp.float32)


def _xs0(h, w):
    return pl.pallas_call(
        _xs0_body,
        grid=(_NP // 512,),
        in_specs=[pl.BlockSpec((512, H), lambda i: (i, 0)),
                  pl.BlockSpec((H, F), lambda i: (0, 0))],
        out_specs=pl.BlockSpec((512, F), lambda i: (i, 0)),
        out_shape=jax.ShapeDtypeStruct((_NP, F), jnp.float32),
    )(h, w)


_RB = 512


def _readout_body(h_ref, b_ref, w1_ref, b1_ref, w2_ref, b2_ref, out_ref):
    i = pl.program_id(0)

    @pl.when(i == 0)
    def _():
        out_ref[...] = jnp.zeros_like(out_ref)

    hb = h_ref[...]
    x = _ssp(jnp.dot(hb, w1_ref[...], preferred_element_type=jnp.float32)
             + b1_ref[...])
    y = jnp.dot(x, w2_ref[...], preferred_element_type=jnp.float32) + b2_ref[...]
    bb = b_ref[0, 0, :]
    gids = jax.lax.broadcasted_iota(jnp.int32, (NG, _RB), 0)
    onehot = (bb[None, :] == gids).astype(jnp.float32)
    out_ref[...] += jnp.dot(onehot, y, preferred_element_type=jnp.float32)


def _readout(h, batch_padded, lin1_w, lin1_b, lin2_w, lin2_b):
    nb = _NP // _RB
    bp = batch_padded.reshape(nb, 1, _RB)
    out = pl.pallas_call(
        _readout_body,
        grid=(nb,),
        in_specs=[
            pl.BlockSpec((_RB, H), lambda i: (i, 0)),
            pl.BlockSpec((1, 1, _RB), lambda i: (i, 0, 0)),
            pl.BlockSpec((H, H // 2), lambda i: (0, 0)),
            pl.BlockSpec((H // 2,), lambda i: (0,)),
            pl.BlockSpec((H // 2, OUT), lambda i: (0, 0)),
            pl.BlockSpec((OUT,), lambda i: (0,)),
        ],
        out_specs=pl.BlockSpec((NG, OUT), lambda i: (0, 0)),
        out_shape=jax.ShapeDtypeStruct((NG, OUT), jnp.float32),
    )(h, bp, lin1_w, lin1_b, lin2_w, lin2_b)
    return out


def kernel(z, pos, batch, emb, iw_mlp1, ib_mlp1, iw_mlp2, ib_mlp2, iw_cl1,
           iw_cl2, ib_cl2, iw_lin, ib_lin, lin1_w, lin1_b, lin2_w, lin2_b,
           target_mean, target_std):
    src, mask = _build_radius_graph(pos, batch)
    dst = jnp.repeat(jnp.arange(N), K)
    diff = pos[dst] - pos[src]
    dist = jnp.sqrt(jnp.maximum(jnp.sum(diff * diff, axis=-1), 1e-12))
    C = 0.5 * (jnp.cos(dist * jnp.pi / CUT) + 1.0) * mask.astype(pos.dtype)

    pad_e = _EP - N * K
    dist_p = jnp.pad(dist, (0, pad_e), constant_values=1.0).reshape(_EP, 1)
    c_p = jnp.pad(C, (0, pad_e)).reshape(_EP, 1)
    src_p = jnp.pad(src, (0, pad_e)).astype(jnp.int32)
    h = jnp.pad(emb[z], ((0, _NP - N), (0, 0)))

    xs = _xs0(h, iw_cl1[0])
    zero_w = jnp.zeros((H, F), jnp.float32)
    for l in range(L):
        xg = _sc_gather(xs, src_p)
        cl1n = iw_cl1[l + 1] if l + 1 < L else zero_w
        h, xs = _layer(dist_p, c_p, xg, h, iw_mlp1[l], ib_mlp1[l],
                       iw_mlp2[l], ib_mlp2[l], iw_cl2[l], ib_cl2[l],
                       iw_lin[l], ib_lin[l], cl1n)

    batch_p = jnp.pad(batch.astype(jnp.int32), (0, _NP - N),
                      constant_values=NG)
    out = _readout(h, batch_p, lin1_w, lin1_b, lin2_w, lin2_b)
    return out * target_std + target_mean


# speedup vs baseline: 1.0025x; 1.0025x over previous
import functools
import math

import jax
import jax.numpy as jnp
from jax import lax
from jax.experimental import pallas as pl
from jax.experimental.pallas import tpu as pltpu
from jax.experimental.pallas import tpu_sc as plsc

N = 10000
NG = 512
H = 128
F = 128
L = 6
G = 50
CUT = 10.0
K = 32
OUT = 4
P = 64

_BN = 256
_BE = _BN * K
_NP = 10240
_EP = _NP * K
_NW = 32
_PER_W = _EP // _NW
_CH = 128
_NCH = _PER_W // _CH


def _ssp(x):
    return jax.nn.softplus(x) - jnp.log(2.0)


def _build_graph_full(pos, batch):
    n = pos.shape[0]
    sq = jnp.sum(pos * pos, axis=-1)
    all_idx = jnp.arange(n)
    srcs, masks = [], []
    block = 1000
    for s in range(0, n, block):
        pb = pos[s:s + block]
        nb = pb.shape[0]
        rows = jnp.arange(s, s + nb)
        d2 = sq[s:s + nb, None] + sq[None, :] - 2.0 * (pb @ pos.T)
        valid = (batch[s:s + nb, None] == batch[None, :]) & (
            rows[:, None] != all_idx[None, :]) & (d2 <= CUT * CUT)
        d2m = jnp.where(valid, d2, jnp.inf)
        vals, idx = jax.lax.top_k(-d2m, K)
        srcs.append(idx.reshape(-1))
        masks.append(jnp.isfinite(vals).reshape(-1))
    return jnp.concatenate(srcs).reshape(N, K), jnp.concatenate(masks).reshape(N, K)


def _build_graph_windowed(pos, batch, seg_start, seg_len):
    slot = jnp.arange(P)
    gidx = seg_start[:, None] + slot[None, :]
    valid_slot = slot[None, :] < seg_len[:, None]
    gidx_c = jnp.where(valid_slot, gidx, N)
    posp = jnp.concatenate([pos, jnp.full((1, 3), 1e9, pos.dtype)], axis=0)
    gpos = posp[gidx_c]
    sq = jnp.sum(gpos * gpos, axis=-1)
    d2 = sq[:, :, None] + sq[:, None, :] - 2.0 * jnp.einsum(
        "gpc,gqc->gpq", gpos, gpos)
    eye = slot[:, None] == slot[None, :]
    valid = valid_slot[:, None, :] & (~eye)[None, :, :] & (d2 <= CUT * CUT)
    d2m = jnp.where(valid, d2, jnp.inf)
    vals, idx = jax.lax.top_k(-d2m.reshape(NG * P, P), K)
    mask_rows = jnp.isfinite(vals)
    src_rows = (seg_start[:, None, None] + idx.reshape(NG, P, K)).reshape(
        NG * P, K)
    src_rows = jnp.where(mask_rows, src_rows, 0)
    rows = batch * P + (jnp.arange(N) - seg_start[batch])
    return src_rows[rows], mask_rows[rows]


def _build_radius_graph(pos, batch):
    pos = jax.lax.stop_gradient(pos)
    batch = batch.astype(jnp.int32)
    gids = jnp.arange(NG, dtype=batch.dtype)
    seg_start = jnp.searchsorted(batch, gids, side="left").astype(jnp.int32)
    seg_end = jnp.searchsorted(batch, gids, side="right").astype(jnp.int32)
    seg_len = seg_end - seg_start
    overflow = jnp.max(seg_len) > P
    src, mask = jax.lax.cond(
        overflow,
        lambda: _build_graph_full(pos, batch),
        lambda: _build_graph_windowed(pos, batch, seg_start, seg_len),
    )
    return src.reshape(-1), mask.reshape(-1)


_SCH = 2
_SROWS = _SCH * _CH
_NSC = _PER_W // _SROWS


def _sc_gather_body(xs_hbm, idx_hbm, out_hbm, idx_v, buf_a, buf_b,
                    gs_a, gs_b):
    c = lax.axis_index("c")
    s = lax.axis_index("s")
    wid = s * 2 + c
    pltpu.sync_copy(idx_hbm.at[wid], idx_v)
    base = wid * _PER_W

    def fire(sc, buf, sem):
        for j in range(_SCH):
            pltpu.async_copy(xs_hbm.at[idx_v.at[sc * _SCH + j]],
                             buf.at[pl.ds(j * _CH, _CH)], sem)

    def drain(buf, sem):
        pltpu.make_async_copy(xs_hbm.at[pl.ds(0, _SROWS)], buf, sem).wait()

    def write(sc, buf):
        pltpu.sync_copy(buf, out_hbm.at[pl.ds(base + sc * _SROWS, _SROWS)])

    fire(0, buf_a, gs_a)

    def step(p, _):
        sc_a = 2 * p
        sc_b = 2 * p + 1
        fire(sc_b, buf_b, gs_b)
        drain(buf_a, gs_a)
        write(sc_a, buf_a)

        @pl.when(sc_a + 2 < _NSC)
        def _():
            fire(sc_a + 2, buf_a, gs_a)

        drain(buf_b, gs_b)
        write(sc_b, buf_b)
        return 0

    lax.fori_loop(0, _NSC // 2, step, 0)


def _sc_gather(xs, src):
    mesh = plsc.VectorSubcoreMesh(core_axis_name="c", subcore_axis_name="s")
    idx3 = src.reshape(_NW, _NCH, _CH)
    f = pl.kernel(
        _sc_gather_body,
        out_type=jax.ShapeDtypeStruct((_EP, H), jnp.float32),
        mesh=mesh,
        scratch_types=[
            pltpu.VMEM((_NCH, _CH), jnp.int32),
            pltpu.VMEM((_SROWS, H), jnp.float32),
            pltpu.VMEM((_SROWS, H), jnp.float32),
            pltpu.SemaphoreType.DMA,
            pltpu.SemaphoreType.DMA,
        ],
    )
    return f(xs, idx3)


_COEFF = -0.5 / (CUT / (G - 1)) ** 2


def _layer_body(d_ref, c_ref, xg_ref, h_ref, w1_ref, b1_ref, w2_ref, b2_ref,
                cl2_ref, bcl2_ref, wlin_ref, blin_ref, cl1n_ref,
                hout_ref, xsout_ref):
    d = d_ref[...]
    offs = lax.broadcasted_iota(jnp.int32, (_BE, G), 1).astype(
        jnp.float32) * (CUT / (G - 1))
    ea = jnp.exp(_COEFF * (d - offs) ** 2)
    t = _ssp(jnp.dot(ea, w1_ref[...],
                     preferred_element_type=jnp.float32) + b1_ref[...])
    w = (jnp.dot(t, w2_ref[...],
                 preferred_element_type=jnp.float32) + b2_ref[...]) * c_ref[...]
    msg = w * xg_ref[...]
    agg = jnp.sum(msg.reshape(_BN, K, H), axis=1)
    conv = jnp.dot(agg, cl2_ref[...],
                   preferred_element_type=jnp.float32) + bcl2_ref[...]
    hn = h_ref[...] + jnp.dot(_ssp(conv), wlin_ref[...],
                              preferred_element_type=jnp.float32) + blin_ref[...]
    hout_ref[...] = hn
    xsout_ref[...] = jnp.dot(hn, cl1n_ref[...],
                             preferred_element_type=jnp.float32)


def _layer(d2d, c2d, xg, h, w1, b1, w2, b2, cl2, bcl2, wlin, blin, cl1n):
    nb = _NP // _BN
    full = lambda i: (0, 0)
    outs = pl.pallas_call(
        _layer_body,
        grid=(nb,),
        in_specs=[
            pl.BlockSpec((_BE, 1), lambda i: (i, 0)),
            pl.BlockSpec((_BE, 1), lambda i: (i, 0)),
            pl.BlockSpec((_BE, H), lambda i: (i, 0)),
            pl.BlockSpec((_BN, H), lambda i: (i, 0)),
            pl.BlockSpec((G, F), full),
            pl.BlockSpec((F,), lambda i: (0,)),
            pl.BlockSpec((F, F), full),
            pl.BlockSpec((F,), lambda i: (0,)),
            pl.BlockSpec((F, H), full),
            pl.BlockSpec((H,), lambda i: (0,)),
            pl.BlockSpec((H, H), full),
            pl.BlockSpec((H,), lambda i: (0,)),
            pl.BlockSpec((H, F), full),
        ],
        out_specs=[
            pl.BlockSpec((_BN, H), lambda i: (i, 0)),
            pl.BlockSpec((_BN, F), lambda i: (i, 0)),
        ],
        out_shape=[
            jax.ShapeDtypeStruct((_NP, H), jnp.float32),
            jax.ShapeDtypeStruct((_NP, F), jnp.float32),
        ],
    )(d2d, c2d, xg, h, w1, b1, w2, b2, cl2, bcl2, wlin, blin, cl1n)
    return outs


def _xs0_body(h_ref, w_ref, out_ref):
    out_ref[...] = jnp.dot(h_ref[...], w_ref[...],
                           preferred_element_type=jnp.float32)


def _xs0(h, w):
    return pl.pallas_call(
        _xs0_body,
        grid=(_NP // 512,),
        in_specs=[pl.BlockSpec((512, H), lambda i: (i, 0)),
                  pl.BlockSpec((H, F), lambda i: (0, 0))],
        out_specs=pl.BlockSpec((512, F), lambda i: (i, 0)),
        out_shape=jax.ShapeDtypeStruct((_NP, F), jnp.float32),
    )(h, w)


_RB = 512


def _readout_body(h_ref, b_ref, w1_ref, b1_ref, w2_ref, b2_ref, out_ref):
    i = pl.program_id(0)

    @pl.when(i == 0)
    def _():
        out_ref[...] = jnp.zeros_like(out_ref)

    hb = h_ref[...]
    x = _ssp(jnp.dot(hb, w1_ref[...], preferred_element_type=jnp.float32)
             + b1_ref[...])
    y = jnp.dot(x, w2_ref[...], preferred_element_type=jnp.float32) + b2_ref[...]
    bb = b_ref[0, 0, :]
    gids = jax.lax.broadcasted_iota(jnp.int32, (NG, _RB), 0)
    onehot = (bb[None, :] == gids).astype(jnp.float32)
    out_ref[...] += jnp.dot(onehot, y, preferred_element_type=jnp.float32)


def _readout(h, batch_padded, lin1_w, lin1_b, lin2_w, lin2_b):
    nb = _NP // _RB
    bp = batch_padded.reshape(nb, 1, _RB)
    out = pl.pallas_call(
        _readout_body,
        grid=(nb,),
        in_specs=[
            pl.BlockSpec((_RB, H), lambda i: (i, 0)),
            pl.BlockSpec((1, 1, _RB), lambda i: (i, 0, 0)),
            pl.BlockSpec((H, H // 2), lambda i: (0, 0)),
            pl.BlockSpec((H // 2,), lambda i: (0,)),
            pl.BlockSpec((H // 2, OUT), lambda i: (0, 0)),
            pl.BlockSpec((OUT,), lambda i: (0,)),
        ],
        out_specs=pl.BlockSpec((NG, OUT), lambda i: (0, 0)),
        out_shape=jax.ShapeDtypeStruct((NG, OUT), jnp.float32),
    )(h, bp, lin1_w, lin1_b, lin2_w, lin2_b)
    return out


def kernel(z, pos, batch, emb, iw_mlp1, ib_mlp1, iw_mlp2, ib_mlp2, iw_cl1,
           iw_cl2, ib_cl2, iw_lin, ib_lin, lin1_w, lin1_b, lin2_w, lin2_b,
           target_mean, target_std):
    src, mask = _build_radius_graph(pos, batch)
    dst = jnp.repeat(jnp.arange(N), K)
    diff = pos[dst] - pos[src]
    dist = jnp.sqrt(jnp.maximum(jnp.sum(diff * diff, axis=-1), 1e-12))
    C = 0.5 * (jnp.cos(dist * jnp.pi / CUT) + 1.0) * mask.astype(pos.dtype)

    pad_e = _EP - N * K
    dist_p = jnp.pad(dist, (0, pad_e), constant_values=1.0).reshape(_EP, 1)
    c_p = jnp.pad(C, (0, pad_e)).reshape(_EP, 1)
    src_p = jnp.pad(src, (0, pad_e)).astype(jnp.int32)
    h = jnp.pad(emb[z], ((0, _NP - N), (0, 0)))

    xs = _xs0(h, iw_cl1[0])
    zero_w = jnp.zeros((H, F), jnp.float32)
    for l in range(L):
        xg = _sc_gather(xs, src_p)
        cl1n = iw_cl1[l + 1] if l + 1 < L else zero_w
        h, xs = _layer(dist_p, c_p, xg, h, iw_mlp1[l], ib_mlp1[l],
                       iw_mlp2[l], ib_mlp2[l], iw_cl2[l], ib_cl2[l],
                       iw_lin[l], ib_lin[l], cl1n)

    batch_p = jnp.pad(batch.astype(jnp.int32), (0, _NP - N),
                      constant_values=NG)
    out = _readout(h, batch_p, lin1_w, lin1_b, lin2_w, lin2_b)
    return out * target_std + target_mean


# speedup vs baseline: 3.0900x; 3.0824x over previous
import functools
import math

import jax
import jax.numpy as jnp
from jax import lax
from jax.experimental import pallas as pl
from jax.experimental.pallas import tpu as pltpu
from jax.experimental.pallas import tpu_sc as plsc

N = 10000
NG = 512
H = 128
F = 128
L = 6
G = 50
CUT = 10.0
K = 32
OUT = 4
P = 64

_BN = 256
_BE = _BN * K
_NP = 10240
_EP = _NP * K
_NW = 32
_NDW = _NP // _NW
_PER_W = _EP // _NW
_CH = 128
_NCH = _PER_W // _CH
_WIN = 456
_USE_SC = False
_AGH = 160


def _ssp(x):
    return jax.nn.softplus(x) - jnp.log(2.0)


def _build_graph_full(pos, batch):
    n = pos.shape[0]
    sq = jnp.sum(pos * pos, axis=-1)
    all_idx = jnp.arange(n)
    srcs, masks = [], []
    block = 1000
    for s in range(0, n, block):
        pb = pos[s:s + block]
        nb = pb.shape[0]
        rows = jnp.arange(s, s + nb)
        d2 = sq[s:s + nb, None] + sq[None, :] - 2.0 * (pb @ pos.T)
        valid = (batch[s:s + nb, None] == batch[None, :]) & (
            rows[:, None] != all_idx[None, :]) & (d2 <= CUT * CUT)
        d2m = jnp.where(valid, d2, jnp.inf)
        vals, idx = jax.lax.top_k(-d2m, K)
        srcs.append(idx.reshape(-1))
        masks.append(jnp.isfinite(vals).reshape(-1))
    return jnp.concatenate(srcs).reshape(N, K), jnp.concatenate(masks).reshape(N, K)


def _build_graph_windowed(pos, batch, seg_start, seg_len):
    slot = jnp.arange(P)
    gidx = seg_start[:, None] + slot[None, :]
    valid_slot = slot[None, :] < seg_len[:, None]
    gidx_c = jnp.where(valid_slot, gidx, N)
    posp = jnp.concatenate([pos, jnp.full((1, 3), 1e9, pos.dtype)], axis=0)
    gpos = posp[gidx_c]
    sq = jnp.sum(gpos * gpos, axis=-1)
    d2 = sq[:, :, None] + sq[:, None, :] - 2.0 * jnp.einsum(
        "gpc,gqc->gpq", gpos, gpos)
    eye = slot[:, None] == slot[None, :]
    valid = valid_slot[:, None, :] & (~eye)[None, :, :] & (d2 <= CUT * CUT)
    d2m = jnp.where(valid, d2, jnp.inf)
    vals, idx = jax.lax.top_k(-d2m.reshape(NG * P, P), K)
    mask_rows = jnp.isfinite(vals)
    src_rows = (seg_start[:, None, None] + idx.reshape(NG, P, K)).reshape(
        NG * P, K)
    src_rows = jnp.where(mask_rows, src_rows, 0)
    rows = batch * P + (jnp.arange(N) - seg_start[batch])
    return src_rows[rows], mask_rows[rows]


_COEFF = -0.5 / (CUT / (G - 1)) ** 2


def _wf_body(d_ref, c_ref, w1_ref, b1_ref, w2_ref, b2_ref, wf_ref):
    d = d_ref[...]
    offs = lax.broadcasted_iota(jnp.int32, (_BE, G), 1).astype(
        jnp.float32) * (CUT / (G - 1))
    ea = jnp.exp(_COEFF * (d - offs) ** 2)
    t = _ssp(jnp.dot(ea, w1_ref[...],
                     preferred_element_type=jnp.float32) + b1_ref[...])
    wf_ref[...] = (jnp.dot(t, w2_ref[...],
                           preferred_element_type=jnp.float32)
                   + b2_ref[...]) * c_ref[...]


def _wf(d2d, c2d, w1, b1, w2, b2):
    nb = _EP // _BE
    full = lambda i: (0, 0)
    return pl.pallas_call(
        _wf_body,
        grid=(nb,),
        in_specs=[
            pl.BlockSpec((_BE, 1), lambda i: (i, 0)),
            pl.BlockSpec((_BE, 1), lambda i: (i, 0)),
            pl.BlockSpec((G, F), full),
            pl.BlockSpec((F,), lambda i: (0,)),
            pl.BlockSpec((F, F), full),
            pl.BlockSpec((F,), lambda i: (0,)),
        ],
        out_specs=pl.BlockSpec((_BE, F), lambda i: (i, 0)),
        out_shape=jax.ShapeDtypeStruct((_EP, F), jnp.float32),
    )(d2d, c2d, w1, b1, w2, b2)


def _sc_msg_body(xs_hbm, wf_hbm, idx_hbm, agg_hbm,
                 win_v, idx_v, wf0, wf1, agg_v, s0, s1):
    c = lax.axis_index("c")
    s = lax.axis_index("s")
    wid = s * 2 + c
    pltpu.sync_copy(idx_hbm.at[wid], idx_v)
    ws = pl.multiple_of(
        jnp.clip(wid * _NDW - P, 0, _NP - _WIN).astype(jnp.int32), 8)
    pltpu.sync_copy(xs_hbm.at[pl.ds(ws * H, _WIN * H)], win_v)
    ebase = wid * _PER_W
    lanes = jnp.arange(16, dtype=jnp.int32)

    def fire(i, buf, sem):
        pltpu.async_copy(wf_hbm.at[pl.ds(ebase + i * _CH, _CH)], buf, sem)

    def drain(i, buf, sem):
        pltpu.make_async_copy(wf_hbm.at[pl.ds(ebase + i * _CH, _CH)],
                              buf, sem).wait()

    fire(0, wf0, s0)
    fire(1, wf1, s1)

    def chunk(i, buf, arow):
        for d4 in range(4):
            acc = tuple(jnp.zeros((16,), jnp.float32) for _ in range(8))
            for half in range(2):
                rh = idx_v[i, pl.ds(d4 * 32 + half * 16, 16)]

                def ebody(k, a):
                    kvec = jnp.full((16, 1), k, jnp.int32)
                    rows = lax.gather(
                        rh, kvec,
                        lax.GatherDimensionNumbers(
                            offset_dims=(), collapsed_slice_dims=(0,),
                            start_index_map=(0,)),
                        (1,),
                        mode=lax.GatherScatterMode.PROMISE_IN_BOUNDS)
                    base_w = rows * H + lanes
                    e = d4 * 32 + half * 16 + k
                    new = []
                    for v in range(8):
                        g = plsc.load_gather(win_v, [base_w + (16 * v)])
                        wrow = buf[e, pl.ds(16 * v, 16)]
                        new.append(a[v] + g * wrow)
                    return tuple(new)

                acc = lax.fori_loop(0, 16, ebody, acc)
            row = arow * 4 + d4
            for v in range(8):
                agg_v[row, pl.ds(16 * v, 16)] = acc[v]

    def do_half(hbase_c, out_row):
        def pbody(p, _):
            i = hbase_c + 2 * p

            drain(i, wf0, s0)
            chunk(i, wf0, 2 * p)

            @pl.when(i + 2 < _NCH)
            def _():
                fire(i + 2, wf0, s0)

            drain(i + 1, wf1, s1)
            chunk(i + 1, wf1, 2 * p + 1)

            @pl.when(i + 3 < _NCH)
            def _():
                fire(i + 3, wf1, s1)

            return 0

        lax.fori_loop(0, _NCH // 4, pbody, 0)
        pltpu.sync_copy(
            agg_v, agg_hbm.at[pl.ds(wid * _NDW + out_row, _AGH)])

    do_half(0, 0)
    do_half(_NCH // 2, _AGH)


def _sc_msg(xs, wf, rel3):
    mesh = plsc.VectorSubcoreMesh(core_axis_name="c", subcore_axis_name="s")
    f = pl.kernel(
        _sc_msg_body,
        out_type=jax.ShapeDtypeStruct((_NP, H), jnp.float32),
        mesh=mesh,
        scratch_types=[
            pltpu.VMEM((_WIN * H,), jnp.float32),
            pltpu.VMEM((_NCH, _CH), jnp.int32),
            pltpu.VMEM((_CH, F), jnp.float32),
            pltpu.VMEM((_CH, F), jnp.float32),
            pltpu.VMEM((_AGH, H), jnp.float32),
            pltpu.SemaphoreType.DMA,
            pltpu.SemaphoreType.DMA,
        ],
    )
    return f(xs.reshape(-1), wf, rel3)


def _post_body(agg_ref, h_ref, cl2_ref, bcl2_ref, wlin_ref, blin_ref,
               cl1n_ref, hout_ref, xsout_ref):
    conv = jnp.dot(agg_ref[...], cl2_ref[...],
                   preferred_element_type=jnp.float32) + bcl2_ref[...]
    hn = h_ref[...] + jnp.dot(_ssp(conv), wlin_ref[...],
                              preferred_element_type=jnp.float32) + blin_ref[...]
    hout_ref[...] = hn
    xsout_ref[...] = jnp.dot(hn, cl1n_ref[...],
                             preferred_element_type=jnp.float32)


def _post(agg, h, cl2, bcl2, wlin, blin, cl1n):
    nb = _NP // 512
    full = lambda i: (0, 0)
    return pl.pallas_call(
        _post_body,
        grid=(nb,),
        in_specs=[
            pl.BlockSpec((512, H), lambda i: (i, 0)),
            pl.BlockSpec((512, H), lambda i: (i, 0)),
            pl.BlockSpec((F, H), full),
            pl.BlockSpec((H,), lambda i: (0,)),
            pl.BlockSpec((H, H), full),
            pl.BlockSpec((H,), lambda i: (0,)),
            pl.BlockSpec((H, F), full),
        ],
        out_specs=[
            pl.BlockSpec((512, H), lambda i: (i, 0)),
            pl.BlockSpec((512, F), lambda i: (i, 0)),
        ],
        out_shape=[
            jax.ShapeDtypeStruct((_NP, H), jnp.float32),
            jax.ShapeDtypeStruct((_NP, F), jnp.float32),
        ],
    )(agg, h, cl2, bcl2, wlin, blin, cl1n)


def _xs0_body(h_ref, w_ref, out_ref):
    out_ref[...] = jnp.dot(h_ref[...], w_ref[...],
                           preferred_element_type=jnp.float32)


def _xs0(h, w):
    return pl.pallas_call(
        _xs0_body,
        grid=(_NP // 512,),
        in_specs=[pl.BlockSpec((512, H), lambda i: (i, 0)),
                  pl.BlockSpec((H, F), lambda i: (0, 0))],
        out_specs=pl.BlockSpec((512, F), lambda i: (i, 0)),
        out_shape=jax.ShapeDtypeStruct((_NP, F), jnp.float32),
    )(h, w)


_RB = 512


def _readout_body(h_ref, b_ref, w1_ref, b1_ref, w2_ref, b2_ref, out_ref):
    i = pl.program_id(0)

    @pl.when(i == 0)
    def _():
        out_ref[...] = jnp.zeros_like(out_ref)

    hb = h_ref[...]
    x = _ssp(jnp.dot(hb, w1_ref[...], preferred_element_type=jnp.float32)
             + b1_ref[...])
    y = jnp.dot(x, w2_ref[...], preferred_element_type=jnp.float32) + b2_ref[...]
    bb = b_ref[0, 0, :]
    gids = jax.lax.broadcasted_iota(jnp.int32, (NG, _RB), 0)
    onehot = (bb[None, :] == gids).astype(jnp.float32)
    out_ref[...] += jnp.dot(onehot, y, preferred_element_type=jnp.float32)


def _readout(h, batch_padded, lin1_w, lin1_b, lin2_w, lin2_b):
    nb = _NP // _RB
    bp = batch_padded.reshape(nb, 1, _RB)
    out = pl.pallas_call(
        _readout_body,
        grid=(nb,),
        in_specs=[
            pl.BlockSpec((_RB, H), lambda i: (i, 0)),
            pl.BlockSpec((1, 1, _RB), lambda i: (i, 0, 0)),
            pl.BlockSpec((H, H // 2), lambda i: (0, 0)),
            pl.BlockSpec((H // 2,), lambda i: (0,)),
            pl.BlockSpec((H // 2, OUT), lambda i: (0, 0)),
            pl.BlockSpec((OUT,), lambda i: (0,)),
        ],
        out_specs=pl.BlockSpec((NG, OUT), lambda i: (0, 0)),
        out_shape=jax.ShapeDtypeStruct((NG, OUT), jnp.float32),
    )(h, bp, lin1_w, lin1_b, lin2_w, lin2_b)
    return out


def kernel(z, pos, batch, emb, iw_mlp1, ib_mlp1, iw_mlp2, ib_mlp2, iw_cl1,
           iw_cl2, ib_cl2, iw_lin, ib_lin, lin1_w, lin1_b, lin2_w, lin2_b,
           target_mean, target_std):
    posg = jax.lax.stop_gradient(pos)
    batch32 = batch.astype(jnp.int32)
    gids = jnp.arange(NG, dtype=jnp.int32)
    seg_start = jnp.searchsorted(batch32, gids, side="left").astype(jnp.int32)
    seg_end = jnp.searchsorted(batch32, gids, side="right").astype(jnp.int32)
    seg_len = seg_end - seg_start
    overflow = jnp.max(seg_len) > P

    src2, mask2 = jax.lax.cond(
        overflow,
        lambda: _build_graph_full(posg, batch32),
        lambda: _build_graph_windowed(posg, batch32, seg_start, seg_len),
    )
    src, mask = src2.reshape(-1), mask2.reshape(-1)

    dst = jnp.repeat(jnp.arange(N), K)
    diff = pos[dst] - pos[src]
    dist = jnp.sqrt(jnp.maximum(jnp.sum(diff * diff, axis=-1), 1e-12))
    C = 0.5 * (jnp.cos(dist * jnp.pi / CUT) + 1.0) * mask.astype(pos.dtype)

    pad_e = _EP - N * K
    dist_p = jnp.pad(dist, (0, pad_e), constant_values=1.0).reshape(_EP, 1)
    c_p = jnp.pad(C, (0, pad_e)).reshape(_EP, 1)
    src_p = jnp.pad(src, (0, pad_e)).astype(jnp.int32)
    h0 = jnp.pad(emb[z], ((0, _NP - N), (0, 0)))

    ws = jnp.clip(jnp.arange(_NW, dtype=jnp.int32) * _NDW - P,
                  0, _NP - _WIN)
    rel = jnp.clip(src_p - jnp.repeat(ws, _PER_W), 0, _WIN - 1)
    rel3 = rel.reshape(_NW, _NCH, _CH)

    def fast_layers():
        h = h0
        xs = _xs0(h, iw_cl1[0])
        zero_w = jnp.zeros((H, F), jnp.float32)
        for l in range(L):
            wf = _wf(dist_p, c_p, iw_mlp1[l], ib_mlp1[l],
                     iw_mlp2[l], ib_mlp2[l])
            agg = _sc_msg(xs, wf, rel3) if _USE_SC else (
                (xs[src_p] * wf).reshape(_NP, K, H).sum(axis=1))
            cl1n = iw_cl1[l + 1] if l + 1 < L else zero_w
            h, xs = _post(agg, h, iw_cl2[l], ib_cl2[l],
                          iw_lin[l], ib_lin[l], cl1n)
        return h

    def slow_layers():
        h = h0[:N]
        offset = jnp.linspace(0.0, CUT, G)
        coeff = -0.5 / (offset[1] - offset[0]) ** 2
        edge_attr = jnp.exp(coeff * (dist[:, None] - offset[None, :]) ** 2)
        for l in range(L):
            Wf = (_ssp(edge_attr @ iw_mlp1[l] + ib_mlp1[l]) @ iw_mlp2[l]
                  + ib_mlp2[l]) * C[:, None]
            xs = h @ iw_cl1[l]
            msg = xs[src] * Wf
            agg = msg.reshape(N, K, H).sum(axis=1)
            conv = agg @ iw_cl2[l] + ib_cl2[l]
            h = h + (_ssp(conv) @ iw_lin[l] + ib_lin[l])
        return jnp.pad(h, ((0, _NP - N), (0, 0)))

    h_fast = fast_layers()
    h_slow = jax.lax.cond(
        overflow, slow_layers,
        lambda: jnp.zeros((_NP, H), jnp.float32))
    h = jnp.where(overflow, h_slow, h_fast)

    batch_p = jnp.pad(batch32, (0, _NP - N), constant_values=NG)
    out = _readout(h, batch_p, lin1_w, lin1_b, lin2_w, lin2_b)
    return out * target_std + target_mean


# speedup vs baseline: 3.4693x; 1.1228x over previous
import functools
import math

import jax
import jax.numpy as jnp
from jax import lax
from jax.experimental import pallas as pl
from jax.experimental.pallas import tpu as pltpu
from jax.experimental.pallas import tpu_sc as plsc

N = 10000
NG = 512
H = 128
F = 128
L = 6
G = 50
CUT = 10.0
K = 32
OUT = 4
P = 64

_BN = 256
_BE = _BN * K
_NP = 10240
_EP = _NP * K
_NW = 32
_NDW = _NP // _NW
_PER_W = _EP // _NW
_CH = 128
_NCH = _PER_W // _CH
_WIN = 456
_USE_SC = False
_AGH = 160


def _ssp(x):
    return jax.nn.softplus(x) - jnp.log(2.0)


def _build_graph_full(pos, batch):
    n = pos.shape[0]
    sq = jnp.sum(pos * pos, axis=-1)
    all_idx = jnp.arange(n)
    srcs, masks = [], []
    block = 1000
    for s in range(0, n, block):
        pb = pos[s:s + block]
        nb = pb.shape[0]
        rows = jnp.arange(s, s + nb)
        d2 = sq[s:s + nb, None] + sq[None, :] - 2.0 * (pb @ pos.T)
        valid = (batch[s:s + nb, None] == batch[None, :]) & (
            rows[:, None] != all_idx[None, :]) & (d2 <= CUT * CUT)
        d2m = jnp.where(valid, d2, jnp.inf)
        vals, idx = jax.lax.top_k(-d2m, K)
        srcs.append(idx.reshape(-1))
        masks.append(jnp.isfinite(vals).reshape(-1))
    return jnp.concatenate(srcs).reshape(N, K), jnp.concatenate(masks).reshape(N, K)


def _build_graph_windowed(pos, batch, seg_start, seg_len):
    slot = jnp.arange(P)
    gidx = seg_start[:, None] + slot[None, :]
    valid_slot = slot[None, :] < seg_len[:, None]
    gidx_c = jnp.where(valid_slot, gidx, N)
    posp = jnp.concatenate([pos, jnp.full((1, 3), 1e9, pos.dtype)], axis=0)
    gpos = posp[gidx_c]
    sq = jnp.sum(gpos * gpos, axis=-1)
    d2 = sq[:, :, None] + sq[:, None, :] - 2.0 * jnp.einsum(
        "gpc,gqc->gpq", gpos, gpos)
    eye = slot[:, None] == slot[None, :]
    valid = valid_slot[:, None, :] & (~eye)[None, :, :] & (d2 <= CUT * CUT)
    d2m = jnp.where(valid, d2, jnp.inf)
    vals, idx = jax.lax.top_k(-d2m.reshape(NG * P, P), K)
    mask_rows = jnp.isfinite(vals)
    src_rows = (seg_start[:, None, None] + idx.reshape(NG, P, K)).reshape(
        NG * P, K)
    src_rows = jnp.where(mask_rows, src_rows, 0)
    rows = batch * P + (jnp.arange(N) - seg_start[batch])
    return src_rows[rows], mask_rows[rows]


_COEFF = -0.5 / (CUT / (G - 1)) ** 2


def _wf_body(d_ref, c_ref, w1_ref, b1_ref, w2_ref, b2_ref, wf_ref):
    d = d_ref[...]
    offs = lax.broadcasted_iota(jnp.int32, (_BE, G), 1).astype(
        jnp.float32) * (CUT / (G - 1))
    ea = jnp.exp(_COEFF * (d - offs) ** 2)
    t = _ssp(jnp.dot(ea, w1_ref[...],
                     preferred_element_type=jnp.float32) + b1_ref[...])
    wf_ref[...] = (jnp.dot(t, w2_ref[...],
                           preferred_element_type=jnp.float32)
                   + b2_ref[...]) * c_ref[...]


def _layer_body(d_ref, c_ref, xg_ref, h_ref, w1_ref, b1_ref, w2_ref, b2_ref,
                cl2_ref, bcl2_ref, wlin_ref, blin_ref, cl1n_ref,
                hout_ref, xsout_ref):
    d = d_ref[...]
    offs = lax.broadcasted_iota(jnp.int32, (_BE, G), 1).astype(
        jnp.float32) * (CUT / (G - 1))
    ea = jnp.exp(_COEFF * (d - offs) ** 2)
    t = _ssp(jnp.dot(ea, w1_ref[...],
                     preferred_element_type=jnp.float32) + b1_ref[...])
    w = (jnp.dot(t, w2_ref[...],
                 preferred_element_type=jnp.float32) + b2_ref[...]) * c_ref[...]
    msg = w * xg_ref[...]
    agg = jnp.sum(msg.reshape(_BN, K, H), axis=1)
    conv = jnp.dot(agg, cl2_ref[...],
                   preferred_element_type=jnp.float32) + bcl2_ref[...]
    hn = h_ref[...] + jnp.dot(_ssp(conv), wlin_ref[...],
                              preferred_element_type=jnp.float32) + blin_ref[...]
    hout_ref[...] = hn
    xsout_ref[...] = jnp.dot(hn, cl1n_ref[...],
                             preferred_element_type=jnp.float32)


def _layer(d2d, c2d, xg, h, w1, b1, w2, b2, cl2, bcl2, wlin, blin, cl1n):
    nb = _NP // _BN
    full = lambda i: (0, 0)
    return pl.pallas_call(
        _layer_body,
        grid=(nb,),
        in_specs=[
            pl.BlockSpec((_BE, 1), lambda i: (i, 0)),
            pl.BlockSpec((_BE, 1), lambda i: (i, 0)),
            pl.BlockSpec((_BE, H), lambda i: (i, 0)),
            pl.BlockSpec((_BN, H), lambda i: (i, 0)),
            pl.BlockSpec((G, F), full),
            pl.BlockSpec((F,), lambda i: (0,)),
            pl.BlockSpec((F, F), full),
            pl.BlockSpec((F,), lambda i: (0,)),
            pl.BlockSpec((F, H), full),
            pl.BlockSpec((H,), lambda i: (0,)),
            pl.BlockSpec((H, H), full),
            pl.BlockSpec((H,), lambda i: (0,)),
            pl.BlockSpec((H, F), full),
        ],
        out_specs=[
            pl.BlockSpec((_BN, H), lambda i: (i, 0)),
            pl.BlockSpec((_BN, F), lambda i: (i, 0)),
        ],
        out_shape=[
            jax.ShapeDtypeStruct((_NP, H), jnp.float32),
            jax.ShapeDtypeStruct((_NP, F), jnp.float32),
        ],
    )(d2d, c2d, xg, h, w1, b1, w2, b2, cl2, bcl2, wlin, blin, cl1n)


def _wf(d2d, c2d, w1, b1, w2, b2):
    nb = _EP // _BE
    full = lambda i: (0, 0)
    return pl.pallas_call(
        _wf_body,
        grid=(nb,),
        in_specs=[
            pl.BlockSpec((_BE, 1), lambda i: (i, 0)),
            pl.BlockSpec((_BE, 1), lambda i: (i, 0)),
            pl.BlockSpec((G, F), full),
            pl.BlockSpec((F,), lambda i: (0,)),
            pl.BlockSpec((F, F), full),
            pl.BlockSpec((F,), lambda i: (0,)),
        ],
        out_specs=pl.BlockSpec((_BE, F), lambda i: (i, 0)),
        out_shape=jax.ShapeDtypeStruct((_EP, F), jnp.float32),
    )(d2d, c2d, w1, b1, w2, b2)


def _sc_msg_body(xs_hbm, wf_hbm, idx_hbm, agg_hbm,
                 win_v, idx_v, wf0, wf1, agg_v, s0, s1):
    c = lax.axis_index("c")
    s = lax.axis_index("s")
    wid = s * 2 + c
    pltpu.sync_copy(idx_hbm.at[wid], idx_v)
    ws = pl.multiple_of(
        jnp.clip(wid * _NDW - P, 0, _NP - _WIN).astype(jnp.int32), 8)
    pltpu.sync_copy(xs_hbm.at[pl.ds(ws * H, _WIN * H)], win_v)
    ebase = wid * _PER_W
    lanes = jnp.arange(16, dtype=jnp.int32)

    def fire(i, buf, sem):
        pltpu.async_copy(wf_hbm.at[pl.ds(ebase + i * _CH, _CH)], buf, sem)

    def drain(i, buf, sem):
        pltpu.make_async_copy(wf_hbm.at[pl.ds(ebase + i * _CH, _CH)],
                              buf, sem).wait()

    fire(0, wf0, s0)
    fire(1, wf1, s1)

    def chunk(i, buf, arow):
        for d4 in range(4):
            acc = tuple(jnp.zeros((16,), jnp.float32) for _ in range(8))
            for half in range(2):
                rh = idx_v[i, pl.ds(d4 * 32 + half * 16, 16)]

                def ebody(k, a):
                    kvec = jnp.full((16, 1), k, jnp.int32)
                    rows = lax.gather(
                        rh, kvec,
                        lax.GatherDimensionNumbers(
                            offset_dims=(), collapsed_slice_dims=(0,),
                            start_index_map=(0,)),
                        (1,),
                        mode=lax.GatherScatterMode.PROMISE_IN_BOUNDS)
                    base_w = rows * H + lanes
                    e = d4 * 32 + half * 16 + k
                    new = []
                    for v in range(8):
                        g = plsc.load_gather(win_v, [base_w + (16 * v)])
                        wrow = buf[e, pl.ds(16 * v, 16)]
                        new.append(a[v] + g * wrow)
                    return tuple(new)

                acc = lax.fori_loop(0, 16, ebody, acc)
            row = arow * 4 + d4
            for v in range(8):
                agg_v[row, pl.ds(16 * v, 16)] = acc[v]

    def do_half(hbase_c, out_row):
        def pbody(p, _):
            i = hbase_c + 2 * p

            drain(i, wf0, s0)
            chunk(i, wf0, 2 * p)

            @pl.when(i + 2 < _NCH)
            def _():
                fire(i + 2, wf0, s0)

            drain(i + 1, wf1, s1)
            chunk(i + 1, wf1, 2 * p + 1)

            @pl.when(i + 3 < _NCH)
            def _():
                fire(i + 3, wf1, s1)

            return 0

        lax.fori_loop(0, _NCH // 4, pbody, 0)
        pltpu.sync_copy(
            agg_v, agg_hbm.at[pl.ds(wid * _NDW + out_row, _AGH)])

    do_half(0, 0)
    do_half(_NCH // 2, _AGH)


def _sc_msg(xs, wf, rel3):
    mesh = plsc.VectorSubcoreMesh(core_axis_name="c", subcore_axis_name="s")
    f = pl.kernel(
        _sc_msg_body,
        out_type=jax.ShapeDtypeStruct((_NP, H), jnp.float32),
        mesh=mesh,
        scratch_types=[
            pltpu.VMEM((_WIN * H,), jnp.float32),
            pltpu.VMEM((_NCH, _CH), jnp.int32),
            pltpu.VMEM((_CH, F), jnp.float32),
            pltpu.VMEM((_CH, F), jnp.float32),
            pltpu.VMEM((_AGH, H), jnp.float32),
            pltpu.SemaphoreType.DMA,
            pltpu.SemaphoreType.DMA,
        ],
    )
    return f(xs.reshape(-1), wf, rel3)


def _post_body(agg_ref, h_ref, cl2_ref, bcl2_ref, wlin_ref, blin_ref,
               cl1n_ref, hout_ref, xsout_ref):
    conv = jnp.dot(agg_ref[...], cl2_ref[...],
                   preferred_element_type=jnp.float32) + bcl2_ref[...]
    hn = h_ref[...] + jnp.dot(_ssp(conv), wlin_ref[...],
                              preferred_element_type=jnp.float32) + blin_ref[...]
    hout_ref[...] = hn
    xsout_ref[...] = jnp.dot(hn, cl1n_ref[...],
                             preferred_element_type=jnp.float32)


def _post(agg, h, cl2, bcl2, wlin, blin, cl1n):
    nb = _NP // 512
    full = lambda i: (0, 0)
    return pl.pallas_call(
        _post_body,
        grid=(nb,),
        in_specs=[
            pl.BlockSpec((512, H), lambda i: (i, 0)),
            pl.BlockSpec((512, H), lambda i: (i, 0)),
            pl.BlockSpec((F, H), full),
            pl.BlockSpec((H,), lambda i: (0,)),
            pl.BlockSpec((H, H), full),
            pl.BlockSpec((H,), lambda i: (0,)),
            pl.BlockSpec((H, F), full),
        ],
        out_specs=[
            pl.BlockSpec((512, H), lambda i: (i, 0)),
            pl.BlockSpec((512, F), lambda i: (i, 0)),
        ],
        out_shape=[
            jax.ShapeDtypeStruct((_NP, H), jnp.float32),
            jax.ShapeDtypeStruct((_NP, F), jnp.float32),
        ],
    )(agg, h, cl2, bcl2, wlin, blin, cl1n)


def _xs0_body(h_ref, w_ref, out_ref):
    out_ref[...] = jnp.dot(h_ref[...], w_ref[...],
                           preferred_element_type=jnp.float32)


def _xs0(h, w):
    return pl.pallas_call(
        _xs0_body,
        grid=(_NP // 512,),
        in_specs=[pl.BlockSpec((512, H), lambda i: (i, 0)),
                  pl.BlockSpec((H, F), lambda i: (0, 0))],
        out_specs=pl.BlockSpec((512, F), lambda i: (i, 0)),
        out_shape=jax.ShapeDtypeStruct((_NP, F), jnp.float32),
    )(h, w)


_RB = 512


def _readout_body(h_ref, b_ref, w1_ref, b1_ref, w2_ref, b2_ref, out_ref):
    i = pl.program_id(0)

    @pl.when(i == 0)
    def _():
        out_ref[...] = jnp.zeros_like(out_ref)

    hb = h_ref[...]
    x = _ssp(jnp.dot(hb, w1_ref[...], preferred_element_type=jnp.float32)
             + b1_ref[...])
    y = jnp.dot(x, w2_ref[...], preferred_element_type=jnp.float32) + b2_ref[...]
    bb = b_ref[0, 0, :]
    gids = jax.lax.broadcasted_iota(jnp.int32, (NG, _RB), 0)
    onehot = (bb[None, :] == gids).astype(jnp.float32)
    out_ref[...] += jnp.dot(onehot, y, preferred_element_type=jnp.float32)


def _readout(h, batch_padded, lin1_w, lin1_b, lin2_w, lin2_b):
    nb = _NP // _RB
    bp = batch_padded.reshape(nb, 1, _RB)
    out = pl.pallas_call(
        _readout_body,
        grid=(nb,),
        in_specs=[
            pl.BlockSpec((_RB, H), lambda i: (i, 0)),
            pl.BlockSpec((1, 1, _RB), lambda i: (i, 0, 0)),
            pl.BlockSpec((H, H // 2), lambda i: (0, 0)),
            pl.BlockSpec((H // 2,), lambda i: (0,)),
            pl.BlockSpec((H // 2, OUT), lambda i: (0, 0)),
            pl.BlockSpec((OUT,), lambda i: (0,)),
        ],
        out_specs=pl.BlockSpec((NG, OUT), lambda i: (0, 0)),
        out_shape=jax.ShapeDtypeStruct((NG, OUT), jnp.float32),
    )(h, bp, lin1_w, lin1_b, lin2_w, lin2_b)
    return out


def kernel(z, pos, batch, emb, iw_mlp1, ib_mlp1, iw_mlp2, ib_mlp2, iw_cl1,
           iw_cl2, ib_cl2, iw_lin, ib_lin, lin1_w, lin1_b, lin2_w, lin2_b,
           target_mean, target_std):
    posg = jax.lax.stop_gradient(pos)
    batch32 = batch.astype(jnp.int32)
    gids = jnp.arange(NG, dtype=jnp.int32)
    seg_start = jnp.searchsorted(batch32, gids, side="left").astype(jnp.int32)
    seg_end = jnp.searchsorted(batch32, gids, side="right").astype(jnp.int32)
    seg_len = seg_end - seg_start
    overflow = jnp.max(seg_len) > P

    src2, mask2 = jax.lax.cond(
        overflow,
        lambda: _build_graph_full(posg, batch32),
        lambda: _build_graph_windowed(posg, batch32, seg_start, seg_len),
    )
    src, mask = src2.reshape(-1), mask2.reshape(-1)

    dst = jnp.repeat(jnp.arange(N), K)
    diff = pos[dst] - pos[src]
    dist = jnp.sqrt(jnp.maximum(jnp.sum(diff * diff, axis=-1), 1e-12))
    C = 0.5 * (jnp.cos(dist * jnp.pi / CUT) + 1.0) * mask.astype(pos.dtype)

    pad_e = _EP - N * K
    dist_p = jnp.pad(dist, (0, pad_e), constant_values=1.0).reshape(_EP, 1)
    c_p = jnp.pad(C, (0, pad_e)).reshape(_EP, 1)
    src_p = jnp.pad(src, (0, pad_e)).astype(jnp.int32)
    h0 = jnp.pad(emb[z], ((0, _NP - N), (0, 0)))

    ws = jnp.clip(jnp.arange(_NW, dtype=jnp.int32) * _NDW - P,
                  0, _NP - _WIN)
    rel = jnp.clip(src_p - jnp.repeat(ws, _PER_W), 0, _WIN - 1)
    rel3 = rel.reshape(_NW, _NCH, _CH)

    def fast_layers():
        h = h0
        xs = _xs0(h, iw_cl1[0])
        zero_w = jnp.zeros((H, F), jnp.float32)
        for l in range(L):
            cl1n = iw_cl1[l + 1] if l + 1 < L else zero_w
            if _USE_SC:
                wf = _wf(dist_p, c_p, iw_mlp1[l], ib_mlp1[l],
                         iw_mlp2[l], ib_mlp2[l])
                agg = _sc_msg(xs, wf, rel3)
                h, xs = _post(agg, h, iw_cl2[l], ib_cl2[l],
                              iw_lin[l], ib_lin[l], cl1n)
            else:
                xg = xs[src_p]
                h, xs = _layer(dist_p, c_p, xg, h, iw_mlp1[l], ib_mlp1[l],
                               iw_mlp2[l], ib_mlp2[l], iw_cl2[l], ib_cl2[l],
                               iw_lin[l], ib_lin[l], cl1n)
        return h

    def slow_layers():
        h = h0[:N]
        offset = jnp.linspace(0.0, CUT, G)
        coeff = -0.5 / (offset[1] - offset[0]) ** 2
        edge_attr = jnp.exp(coeff * (dist[:, None] - offset[None, :]) ** 2)
        for l in range(L):
            Wf = (_ssp(edge_attr @ iw_mlp1[l] + ib_mlp1[l]) @ iw_mlp2[l]
                  + ib_mlp2[l]) * C[:, None]
            xs = h @ iw_cl1[l]
            msg = xs[src] * Wf
            agg = msg.reshape(N, K, H).sum(axis=1)
            conv = agg @ iw_cl2[l] + ib_cl2[l]
            h = h + (_ssp(conv) @ iw_lin[l] + ib_lin[l])
        return jnp.pad(h, ((0, _NP - N), (0, 0)))

    h_fast = fast_layers()
    h_slow = jax.lax.cond(
        overflow, slow_layers,
        lambda: jnp.zeros((_NP, H), jnp.float32))
    h = jnp.where(overflow, h_slow, h_fast)

    batch_p = jnp.pad(batch32, (0, _NP - N), constant_values=NG)
    out = _readout(h, batch_p, lin1_w, lin1_b, lin2_w, lin2_b)
    return out * target_std + target_mean


# speedup vs baseline: 8.6371x; 2.4895x over previous
import functools
import math

import jax
import jax.numpy as jnp
from jax import lax
from jax.experimental import pallas as pl
from jax.experimental.pallas import tpu as pltpu
from jax.experimental.pallas import tpu_sc as plsc

N = 10000
NG = 512
H = 128
F = 128
L = 6
G = 50
CUT = 10.0
K = 32
OUT = 4
P = 64

_BN = 256
_BE = _BN * K
_NP = 10240
_EP = _NP * K
_NW = 32
_NDW = _NP // _NW
_PER_W = _EP // _NW
_CH = 128
_NCH = _PER_W // _CH
_WIN = 456
_USE_SC = True
_AGH = 160


def _ssp(x):
    return jax.nn.softplus(x) - jnp.log(2.0)


def _build_graph_full(pos, batch):
    n = pos.shape[0]
    sq = jnp.sum(pos * pos, axis=-1)
    all_idx = jnp.arange(n)
    srcs, masks = [], []
    block = 1000
    for s in range(0, n, block):
        pb = pos[s:s + block]
        nb = pb.shape[0]
        rows = jnp.arange(s, s + nb)
        d2 = sq[s:s + nb, None] + sq[None, :] - 2.0 * (pb @ pos.T)
        valid = (batch[s:s + nb, None] == batch[None, :]) & (
            rows[:, None] != all_idx[None, :]) & (d2 <= CUT * CUT)
        d2m = jnp.where(valid, d2, jnp.inf)
        vals, idx = jax.lax.top_k(-d2m, K)
        srcs.append(idx.reshape(-1))
        masks.append(jnp.isfinite(vals).reshape(-1))
    return jnp.concatenate(srcs).reshape(N, K), jnp.concatenate(masks).reshape(N, K)


def _build_graph_windowed(pos, batch, seg_start, seg_len):
    slot = jnp.arange(P)
    gidx = seg_start[:, None] + slot[None, :]
    valid_slot = slot[None, :] < seg_len[:, None]
    gidx_c = jnp.where(valid_slot, gidx, N)
    posp = jnp.concatenate([pos, jnp.full((1, 3), 1e9, pos.dtype)], axis=0)
    gpos = posp[gidx_c]
    sq = jnp.sum(gpos * gpos, axis=-1)
    d2 = sq[:, :, None] + sq[:, None, :] - 2.0 * jnp.einsum(
        "gpc,gqc->gpq", gpos, gpos)
    eye = slot[:, None] == slot[None, :]
    valid = valid_slot[:, None, :] & (~eye)[None, :, :] & (d2 <= CUT * CUT)
    d2m = jnp.where(valid, d2, jnp.inf)
    vals, idx = jax.lax.top_k(-d2m.reshape(NG * P, P), K)
    mask_rows = jnp.isfinite(vals)
    src_rows = (seg_start[:, None, None] + idx.reshape(NG, P, K)).reshape(
        NG * P, K)
    src_rows = jnp.where(mask_rows, src_rows, 0)
    rows = batch * P + (jnp.arange(N) - seg_start[batch])
    return src_rows[rows], mask_rows[rows]


_COEFF = -0.5 / (CUT / (G - 1)) ** 2


def _wf_body(d_ref, c_ref, w1_ref, b1_ref, w2_ref, b2_ref, wf_ref):
    d = d_ref[...]
    offs = lax.broadcasted_iota(jnp.int32, (_BE, G), 1).astype(
        jnp.float32) * (CUT / (G - 1))
    ea = jnp.exp(_COEFF * (d - offs) ** 2)
    t = _ssp(jnp.dot(ea, w1_ref[...],
                     preferred_element_type=jnp.float32) + b1_ref[...])
    wf_ref[...] = (jnp.dot(t, w2_ref[...],
                           preferred_element_type=jnp.float32)
                   + b2_ref[...]) * c_ref[...]


def _layer_body(d_ref, c_ref, xg_ref, h_ref, w1_ref, b1_ref, w2_ref, b2_ref,
                cl2_ref, bcl2_ref, wlin_ref, blin_ref, cl1n_ref,
                hout_ref, xsout_ref):
    d = d_ref[...]
    offs = lax.broadcasted_iota(jnp.int32, (_BE, G), 1).astype(
        jnp.float32) * (CUT / (G - 1))
    ea = jnp.exp(_COEFF * (d - offs) ** 2)
    t = _ssp(jnp.dot(ea, w1_ref[...],
                     preferred_element_type=jnp.float32) + b1_ref[...])
    w = (jnp.dot(t, w2_ref[...],
                 preferred_element_type=jnp.float32) + b2_ref[...]) * c_ref[...]
    msg = w * xg_ref[...]
    agg = jnp.sum(msg.reshape(_BN, K, H), axis=1)
    conv = jnp.dot(agg, cl2_ref[...],
                   preferred_element_type=jnp.float32) + bcl2_ref[...]
    hn = h_ref[...] + jnp.dot(_ssp(conv), wlin_ref[...],
                              preferred_element_type=jnp.float32) + blin_ref[...]
    hout_ref[...] = hn
    xsout_ref[...] = jnp.dot(hn, cl1n_ref[...],
                             preferred_element_type=jnp.float32)


def _layer(d2d, c2d, xg, h, w1, b1, w2, b2, cl2, bcl2, wlin, blin, cl1n):
    nb = _NP // _BN
    full = lambda i: (0, 0)
    return pl.pallas_call(
        _layer_body,
        grid=(nb,),
        in_specs=[
            pl.BlockSpec((_BE, 1), lambda i: (i, 0)),
            pl.BlockSpec((_BE, 1), lambda i: (i, 0)),
            pl.BlockSpec((_BE, H), lambda i: (i, 0)),
            pl.BlockSpec((_BN, H), lambda i: (i, 0)),
            pl.BlockSpec((G, F), full),
            pl.BlockSpec((F,), lambda i: (0,)),
            pl.BlockSpec((F, F), full),
            pl.BlockSpec((F,), lambda i: (0,)),
            pl.BlockSpec((F, H), full),
            pl.BlockSpec((H,), lambda i: (0,)),
            pl.BlockSpec((H, H), full),
            pl.BlockSpec((H,), lambda i: (0,)),
            pl.BlockSpec((H, F), full),
        ],
        out_specs=[
            pl.BlockSpec((_BN, H), lambda i: (i, 0)),
            pl.BlockSpec((_BN, F), lambda i: (i, 0)),
        ],
        out_shape=[
            jax.ShapeDtypeStruct((_NP, H), jnp.float32),
            jax.ShapeDtypeStruct((_NP, F), jnp.float32),
        ],
    )(d2d, c2d, xg, h, w1, b1, w2, b2, cl2, bcl2, wlin, blin, cl1n)


def _wf(d2d, c2d, w1, b1, w2, b2):
    nb = _EP // _BE
    full = lambda i: (0, 0)
    return pl.pallas_call(
        _wf_body,
        grid=(nb,),
        in_specs=[
            pl.BlockSpec((_BE, 1), lambda i: (i, 0)),
            pl.BlockSpec((_BE, 1), lambda i: (i, 0)),
            pl.BlockSpec((G, F), full),
            pl.BlockSpec((F,), lambda i: (0,)),
            pl.BlockSpec((F, F), full),
            pl.BlockSpec((F,), lambda i: (0,)),
        ],
        out_specs=pl.BlockSpec((_BE, F), lambda i: (i, 0)),
        out_shape=jax.ShapeDtypeStruct((_EP, F), jnp.float32),
    )(d2d, c2d, w1, b1, w2, b2)


def _sc_msg_body(xs_hbm, wf_hbm, idx_hbm, agg_hbm,
                 win_sh, idx_v, wf0, xg0, agg_v, s0, s1):
    c = lax.axis_index("c")
    s = lax.axis_index("s")
    wid = s * 2 + c
    pltpu.sync_copy(idx_hbm.at[wid], idx_v)
    ws = pl.multiple_of(
        jnp.clip(wid * _NDW - P, 0, _NP - _WIN).astype(jnp.int32), 8)
    win = win_sh.at[s]
    pltpu.sync_copy(xs_hbm.at[pl.ds(ws, _WIN)], win)
    ebase = wid * _PER_W

    def chunk(i, arow):
        pltpu.async_copy(wf_hbm.at[pl.ds(ebase + i * _CH, _CH)], wf0, s0)
        pltpu.async_copy(win.at[idx_v.at[i]], xg0, s1)
        pltpu.make_async_copy(wf_hbm.at[pl.ds(0, _CH)], wf0, s0).wait()
        pltpu.make_async_copy(win.at[idx_v.at[i]], xg0, s1).wait()
        for d4 in range(4):
            acc = tuple(jnp.zeros((16,), jnp.float32) for _ in range(8))

            def ebody(k, a):
                e = d4 * 32 + k
                new = []
                for v in range(8):
                    g = xg0[e, pl.ds(16 * v, 16)]
                    wrow = wf0[e, pl.ds(16 * v, 16)]
                    new.append(a[v] + g * wrow)
                return tuple(new)

            acc = lax.fori_loop(0, 32, ebody, acc)
            row = arow * 4 + d4
            for v in range(8):
                agg_v[row, pl.ds(16 * v, 16)] = acc[v]

    def do_half(hbase_c, out_row):
        def pbody(p, _):
            chunk(hbase_c + p, p)
            return 0

        lax.fori_loop(0, _NCH // 2, pbody, 0)
        pltpu.sync_copy(
            agg_v, agg_hbm.at[pl.ds(wid * _NDW + out_row, _AGH)])

    do_half(0, 0)
    do_half(_NCH // 2, _AGH)


def _sc_msg(xs, wf, rel3):
    mesh = plsc.VectorSubcoreMesh(core_axis_name="c", subcore_axis_name="s")
    f = pl.kernel(
        _sc_msg_body,
        out_type=jax.ShapeDtypeStruct((_NP, H), jnp.float32),
        mesh=mesh,
        scratch_types=[
            pltpu.VMEM_SHARED((16, _WIN, H), jnp.float32),
            pltpu.VMEM((_NCH, _CH), jnp.int32),
            pltpu.VMEM((_CH, F), jnp.float32),
            pltpu.VMEM((_CH, F), jnp.float32),
            pltpu.VMEM((_AGH, H), jnp.float32),
            pltpu.SemaphoreType.DMA,
            pltpu.SemaphoreType.DMA,
        ],
    )
    return f(xs, wf, rel3)


def _post_body(agg_ref, h_ref, cl2_ref, bcl2_ref, wlin_ref, blin_ref,
               cl1n_ref, hout_ref, xsout_ref):
    conv = jnp.dot(agg_ref[...], cl2_ref[...],
                   preferred_element_type=jnp.float32) + bcl2_ref[...]
    hn = h_ref[...] + jnp.dot(_ssp(conv), wlin_ref[...],
                              preferred_element_type=jnp.float32) + blin_ref[...]
    hout_ref[...] = hn
    xsout_ref[...] = jnp.dot(hn, cl1n_ref[...],
                             preferred_element_type=jnp.float32)


def _post(agg, h, cl2, bcl2, wlin, blin, cl1n):
    nb = _NP // 512
    full = lambda i: (0, 0)
    return pl.pallas_call(
        _post_body,
        grid=(nb,),
        in_specs=[
            pl.BlockSpec((512, H), lambda i: (i, 0)),
            pl.BlockSpec((512, H), lambda i: (i, 0)),
            pl.BlockSpec((F, H), full),
            pl.BlockSpec((H,), lambda i: (0,)),
            pl.BlockSpec((H, H), full),
            pl.BlockSpec((H,), lambda i: (0,)),
            pl.BlockSpec((H, F), full),
        ],
        out_specs=[
            pl.BlockSpec((512, H), lambda i: (i, 0)),
            pl.BlockSpec((512, F), lambda i: (i, 0)),
        ],
        out_shape=[
            jax.ShapeDtypeStruct((_NP, H), jnp.float32),
            jax.ShapeDtypeStruct((_NP, F), jnp.float32),
        ],
    )(agg, h, cl2, bcl2, wlin, blin, cl1n)


def _xs0_body(h_ref, w_ref, out_ref):
    out_ref[...] = jnp.dot(h_ref[...], w_ref[...],
                           preferred_element_type=jnp.float32)


def _xs0(h, w):
    return pl.pallas_call(
        _xs0_body,
        grid=(_NP // 512,),
        in_specs=[pl.BlockSpec((512, H), lambda i: (i, 0)),
                  pl.BlockSpec((H, F), lambda i: (0, 0))],
        out_specs=pl.BlockSpec((512, F), lambda i: (i, 0)),
        out_shape=jax.ShapeDtypeStruct((_NP, F), jnp.float32),
    )(h, w)


_RB = 512


def _readout_body(h_ref, b_ref, w1_ref, b1_ref, w2_ref, b2_ref, out_ref):
    i = pl.program_id(0)

    @pl.when(i == 0)
    def _():
        out_ref[...] = jnp.zeros_like(out_ref)

    hb = h_ref[...]
    x = _ssp(jnp.dot(hb, w1_ref[...], preferred_element_type=jnp.float32)
             + b1_ref[...])
    y = jnp.dot(x, w2_ref[...], preferred_element_type=jnp.float32) + b2_ref[...]
    bb = b_ref[0, 0, :]
    gids = jax.lax.broadcasted_iota(jnp.int32, (NG, _RB), 0)
    onehot = (bb[None, :] == gids).astype(jnp.float32)
    out_ref[...] += jnp.dot(onehot, y, preferred_element_type=jnp.float32)


def _readout(h, batch_padded, lin1_w, lin1_b, lin2_w, lin2_b):
    nb = _NP // _RB
    bp = batch_padded.reshape(nb, 1, _RB)
    out = pl.pallas_call(
        _readout_body,
        grid=(nb,),
        in_specs=[
            pl.BlockSpec((_RB, H), lambda i: (i, 0)),
            pl.BlockSpec((1, 1, _RB), lambda i: (i, 0, 0)),
            pl.BlockSpec((H, H // 2), lambda i: (0, 0)),
            pl.BlockSpec((H // 2,), lambda i: (0,)),
            pl.BlockSpec((H // 2, OUT), lambda i: (0, 0)),
            pl.BlockSpec((OUT,), lambda i: (0,)),
        ],
        out_specs=pl.BlockSpec((NG, OUT), lambda i: (0, 0)),
        out_shape=jax.ShapeDtypeStruct((NG, OUT), jnp.float32),
    )(h, bp, lin1_w, lin1_b, lin2_w, lin2_b)
    return out


def kernel(z, pos, batch, emb, iw_mlp1, ib_mlp1, iw_mlp2, ib_mlp2, iw_cl1,
           iw_cl2, ib_cl2, iw_lin, ib_lin, lin1_w, lin1_b, lin2_w, lin2_b,
           target_mean, target_std):
    posg = jax.lax.stop_gradient(pos)
    batch32 = batch.astype(jnp.int32)
    gids = jnp.arange(NG, dtype=jnp.int32)
    seg_start = jnp.searchsorted(batch32, gids, side="left").astype(jnp.int32)
    seg_end = jnp.searchsorted(batch32, gids, side="right").astype(jnp.int32)
    seg_len = seg_end - seg_start
    overflow = jnp.max(seg_len) > P

    src2, mask2 = jax.lax.cond(
        overflow,
        lambda: _build_graph_full(posg, batch32),
        lambda: _build_graph_windowed(posg, batch32, seg_start, seg_len),
    )
    src, mask = src2.reshape(-1), mask2.reshape(-1)

    dst = jnp.repeat(jnp.arange(N), K)
    diff = pos[dst] - pos[src]
    dist = jnp.sqrt(jnp.maximum(jnp.sum(diff * diff, axis=-1), 1e-12))
    C = 0.5 * (jnp.cos(dist * jnp.pi / CUT) + 1.0) * mask.astype(pos.dtype)

    pad_e = _EP - N * K
    dist_p = jnp.pad(dist, (0, pad_e), constant_values=1.0).reshape(_EP, 1)
    c_p = jnp.pad(C, (0, pad_e)).reshape(_EP, 1)
    src_p = jnp.pad(src, (0, pad_e)).astype(jnp.int32)
    h0 = jnp.pad(emb[z], ((0, _NP - N), (0, 0)))

    ws = jnp.clip(jnp.arange(_NW, dtype=jnp.int32) * _NDW - P,
                  0, _NP - _WIN)
    rel = jnp.clip(src_p - jnp.repeat(ws, _PER_W), 0, _WIN - 1)
    rel3 = rel.reshape(_NW, _NCH, _CH)

    def fast_layers():
        h = h0
        xs = _xs0(h, iw_cl1[0])
        zero_w = jnp.zeros((H, F), jnp.float32)
        for l in range(L):
            cl1n = iw_cl1[l + 1] if l + 1 < L else zero_w
            if _USE_SC:
                wf = _wf(dist_p, c_p, iw_mlp1[l], ib_mlp1[l],
                         iw_mlp2[l], ib_mlp2[l])
                agg = _sc_msg(xs, wf, rel3)
                h, xs = _post(agg, h, iw_cl2[l], ib_cl2[l],
                              iw_lin[l], ib_lin[l], cl1n)
            else:
                xg = xs[src_p]
                h, xs = _layer(dist_p, c_p, xg, h, iw_mlp1[l], ib_mlp1[l],
                               iw_mlp2[l], ib_mlp2[l], iw_cl2[l], ib_cl2[l],
                               iw_lin[l], ib_lin[l], cl1n)
        return h

    def slow_layers():
        h = h0[:N]
        offset = jnp.linspace(0.0, CUT, G)
        coeff = -0.5 / (offset[1] - offset[0]) ** 2
        edge_attr = jnp.exp(coeff * (dist[:, None] - offset[None, :]) ** 2)
        for l in range(L):
            Wf = (_ssp(edge_attr @ iw_mlp1[l] + ib_mlp1[l]) @ iw_mlp2[l]
                  + ib_mlp2[l]) * C[:, None]
            xs = h @ iw_cl1[l]
            msg = xs[src] * Wf
            agg = msg.reshape(N, K, H).sum(axis=1)
            conv = agg @ iw_cl2[l] + ib_cl2[l]
            h = h + (_ssp(conv) @ iw_lin[l] + ib_lin[l])
        return jnp.pad(h, ((0, _NP - N), (0, 0)))

    h_fast = fast_layers()
    h_slow = jax.lax.cond(
        overflow, slow_layers,
        lambda: jnp.zeros((_NP, H), jnp.float32))
    h = jnp.where(overflow, h_slow, h_fast)

    batch_p = jnp.pad(batch32, (0, _NP - N), constant_values=NG)
    out = _readout(h, batch_p, lin1_w, lin1_b, lin2_w, lin2_b)
    return out * target_std + target_mean


# speedup vs baseline: 9.1023x; 1.0539x over previous
import functools
import math

import jax
import jax.numpy as jnp
from jax import lax
from jax.experimental import pallas as pl
from jax.experimental.pallas import tpu as pltpu
from jax.experimental.pallas import tpu_sc as plsc

N = 10000
NG = 512
H = 128
F = 128
L = 6
G = 50
CUT = 10.0
K = 32
OUT = 4
P = 64

_BN = 256
_BE = _BN * K
_NP = 10240
_EP = _NP * K
_NW = 32
_NDW = _NP // _NW
_PER_W = _EP // _NW
_CH = 128
_NCH = _PER_W // _CH
_WIN = 456
_USE_SC = True
_AGH = 80


def _ssp(x):
    return jax.nn.softplus(x) - jnp.log(2.0)


def _build_graph_full(pos, batch):
    n = pos.shape[0]
    sq = jnp.sum(pos * pos, axis=-1)
    all_idx = jnp.arange(n)
    srcs, masks = [], []
    block = 1000
    for s in range(0, n, block):
        pb = pos[s:s + block]
        nb = pb.shape[0]
        rows = jnp.arange(s, s + nb)
        d2 = sq[s:s + nb, None] + sq[None, :] - 2.0 * (pb @ pos.T)
        valid = (batch[s:s + nb, None] == batch[None, :]) & (
            rows[:, None] != all_idx[None, :]) & (d2 <= CUT * CUT)
        d2m = jnp.where(valid, d2, jnp.inf)
        vals, idx = jax.lax.top_k(-d2m, K)
        srcs.append(idx.reshape(-1))
        masks.append(jnp.isfinite(vals).reshape(-1))
    return jnp.concatenate(srcs).reshape(N, K), jnp.concatenate(masks).reshape(N, K)


def _build_graph_windowed(pos, batch, seg_start, seg_len):
    slot = jnp.arange(P)
    gidx = seg_start[:, None] + slot[None, :]
    valid_slot = slot[None, :] < seg_len[:, None]
    gidx_c = jnp.where(valid_slot, gidx, N)
    posp = jnp.concatenate([pos, jnp.full((1, 3), 1e9, pos.dtype)], axis=0)
    gpos = posp[gidx_c]
    sq = jnp.sum(gpos * gpos, axis=-1)
    d2 = sq[:, :, None] + sq[:, None, :] - 2.0 * jnp.einsum(
        "gpc,gqc->gpq", gpos, gpos)
    eye = slot[:, None] == slot[None, :]
    valid = valid_slot[:, None, :] & (~eye)[None, :, :] & (d2 <= CUT * CUT)
    d2m = jnp.where(valid, d2, jnp.inf)
    vals, idx = jax.lax.top_k(-d2m.reshape(NG * P, P), K)
    mask_rows = jnp.isfinite(vals)
    src_rows = (seg_start[:, None, None] + idx.reshape(NG, P, K)).reshape(
        NG * P, K)
    src_rows = jnp.where(mask_rows, src_rows, 0)
    rows = batch * P + (jnp.arange(N) - seg_start[batch])
    return src_rows[rows], mask_rows[rows]


_COEFF = -0.5 / (CUT / (G - 1)) ** 2


def _wf_body(d_ref, c_ref, w1_ref, b1_ref, w2_ref, b2_ref, wf_ref):
    d = d_ref[...]
    offs = lax.broadcasted_iota(jnp.int32, (_BE, G), 1).astype(
        jnp.float32) * (CUT / (G - 1))
    ea = jnp.exp(_COEFF * (d - offs) ** 2)
    t = _ssp(jnp.dot(ea, w1_ref[...],
                     preferred_element_type=jnp.float32) + b1_ref[...])
    wf_ref[...] = (jnp.dot(t, w2_ref[...],
                           preferred_element_type=jnp.float32)
                   + b2_ref[...]) * c_ref[...]


def _layer_body(d_ref, c_ref, xg_ref, h_ref, w1_ref, b1_ref, w2_ref, b2_ref,
                cl2_ref, bcl2_ref, wlin_ref, blin_ref, cl1n_ref,
                hout_ref, xsout_ref):
    d = d_ref[...]
    offs = lax.broadcasted_iota(jnp.int32, (_BE, G), 1).astype(
        jnp.float32) * (CUT / (G - 1))
    ea = jnp.exp(_COEFF * (d - offs) ** 2)
    t = _ssp(jnp.dot(ea, w1_ref[...],
                     preferred_element_type=jnp.float32) + b1_ref[...])
    w = (jnp.dot(t, w2_ref[...],
                 preferred_element_type=jnp.float32) + b2_ref[...]) * c_ref[...]
    msg = w * xg_ref[...]
    agg = jnp.sum(msg.reshape(_BN, K, H), axis=1)
    conv = jnp.dot(agg, cl2_ref[...],
                   preferred_element_type=jnp.float32) + bcl2_ref[...]
    hn = h_ref[...] + jnp.dot(_ssp(conv), wlin_ref[...],
                              preferred_element_type=jnp.float32) + blin_ref[...]
    hout_ref[...] = hn
    xsout_ref[...] = jnp.dot(hn, cl1n_ref[...],
                             preferred_element_type=jnp.float32)


def _layer(d2d, c2d, xg, h, w1, b1, w2, b2, cl2, bcl2, wlin, blin, cl1n):
    nb = _NP // _BN
    full = lambda i: (0, 0)
    return pl.pallas_call(
        _layer_body,
        grid=(nb,),
        in_specs=[
            pl.BlockSpec((_BE, 1), lambda i: (i, 0)),
            pl.BlockSpec((_BE, 1), lambda i: (i, 0)),
            pl.BlockSpec((_BE, H), lambda i: (i, 0)),
            pl.BlockSpec((_BN, H), lambda i: (i, 0)),
            pl.BlockSpec((G, F), full),
            pl.BlockSpec((F,), lambda i: (0,)),
            pl.BlockSpec((F, F), full),
            pl.BlockSpec((F,), lambda i: (0,)),
            pl.BlockSpec((F, H), full),
            pl.BlockSpec((H,), lambda i: (0,)),
            pl.BlockSpec((H, H), full),
            pl.BlockSpec((H,), lambda i: (0,)),
            pl.BlockSpec((H, F), full),
        ],
        out_specs=[
            pl.BlockSpec((_BN, H), lambda i: (i, 0)),
            pl.BlockSpec((_BN, F), lambda i: (i, 0)),
        ],
        out_shape=[
            jax.ShapeDtypeStruct((_NP, H), jnp.float32),
            jax.ShapeDtypeStruct((_NP, F), jnp.float32),
        ],
    )(d2d, c2d, xg, h, w1, b1, w2, b2, cl2, bcl2, wlin, blin, cl1n)


def _wf(d2d, c2d, w1, b1, w2, b2):
    nb = _EP // _BE
    full = lambda i: (0, 0)
    return pl.pallas_call(
        _wf_body,
        grid=(nb,),
        in_specs=[
            pl.BlockSpec((_BE, 1), lambda i: (i, 0)),
            pl.BlockSpec((_BE, 1), lambda i: (i, 0)),
            pl.BlockSpec((G, F), full),
            pl.BlockSpec((F,), lambda i: (0,)),
            pl.BlockSpec((F, F), full),
            pl.BlockSpec((F,), lambda i: (0,)),
        ],
        out_specs=pl.BlockSpec((_BE, F), lambda i: (i, 0)),
        out_shape=jax.ShapeDtypeStruct((_EP, F), jnp.float32),
    )(d2d, c2d, w1, b1, w2, b2)


def _sc_msg_body(xs_hbm, wf_hbm, idx_hbm, agg_hbm,
                 win_sh, idx_v, wf0, xg0, wf1, agg_v, s0, s1, s2):
    c = lax.axis_index("c")
    s = lax.axis_index("s")
    wid = s * 2 + c
    pltpu.sync_copy(idx_hbm.at[wid], idx_v)
    ws = pl.multiple_of(
        jnp.clip(wid * _NDW - P, 0, _NP - _WIN).astype(jnp.int32), 8)
    win = win_sh.at[s]
    pltpu.sync_copy(xs_hbm.at[pl.ds(ws, _WIN)], win)
    ebase = wid * _PER_W

    def fire(i, wfb, sw):
        pltpu.async_copy(wf_hbm.at[pl.ds(ebase + i * _CH, _CH)], wfb, sw)

    def drain(wfb, sw):
        pltpu.make_async_copy(wf_hbm.at[pl.ds(0, _CH)], wfb, sw).wait()

    def gather(i):
        pltpu.async_copy(win.at[idx_v.at[i]], xg0, s1)
        pltpu.make_async_copy(win.at[idx_v.at[i]], xg0, s1).wait()

    def compute(wfb, xgb, arow):
        for d4 in range(4):
            acc = tuple(jnp.zeros((16,), jnp.float32) for _ in range(8))

            def ebody(k, a):
                e = d4 * 32 + k
                new = []
                for v in range(8):
                    g = xgb[e, pl.ds(16 * v, 16)]
                    wrow = wfb[e, pl.ds(16 * v, 16)]
                    new.append(a[v] + g * wrow)
                return tuple(new)

            acc = lax.fori_loop(0, 32, ebody, acc)
            row = arow * 4 + d4
            for v in range(8):
                agg_v[row, pl.ds(16 * v, 16)] = acc[v]

    fire(0, wf0, s0)

    def do_half(hbase_c, out_row):
        def pbody(p, _):
            i = hbase_c + 2 * p
            fire(i + 1, wf1, s2)
            gather(i)
            drain(wf0, s0)
            compute(wf0, xg0, 2 * p)

            @pl.when(i + 2 < _NCH)
            def _():
                fire(i + 2, wf0, s0)

            gather(i + 1)
            drain(wf1, s2)
            compute(wf1, xg0, 2 * p + 1)
            return 0

        lax.fori_loop(0, _NCH // 8, pbody, 0)
        pltpu.sync_copy(
            agg_v, agg_hbm.at[pl.ds(wid * _NDW + out_row, _AGH)])

    for q in range(4):
        do_half(q * (_NCH // 4), q * _AGH)


def _sc_msg(xs, wf, rel3):
    mesh = plsc.VectorSubcoreMesh(core_axis_name="c", subcore_axis_name="s")
    f = pl.kernel(
        _sc_msg_body,
        out_type=jax.ShapeDtypeStruct((_NP, H), jnp.float32),
        mesh=mesh,
        scratch_types=[
            pltpu.VMEM_SHARED((16, _WIN, H), jnp.float32),
            pltpu.VMEM((_NCH, _CH), jnp.int32),
            pltpu.VMEM((_CH, F), jnp.float32),
            pltpu.VMEM((_CH, F), jnp.float32),
            pltpu.VMEM((_CH, F), jnp.float32),
            pltpu.VMEM((_AGH, H), jnp.float32),
            pltpu.SemaphoreType.DMA,
            pltpu.SemaphoreType.DMA,
            pltpu.SemaphoreType.DMA,
        ],
    )
    return f(xs, wf, rel3)


def _post_body(agg_ref, h_ref, cl2_ref, bcl2_ref, wlin_ref, blin_ref,
               cl1n_ref, hout_ref, xsout_ref):
    conv = jnp.dot(agg_ref[...], cl2_ref[...],
                   preferred_element_type=jnp.float32) + bcl2_ref[...]
    hn = h_ref[...] + jnp.dot(_ssp(conv), wlin_ref[...],
                              preferred_element_type=jnp.float32) + blin_ref[...]
    hout_ref[...] = hn
    xsout_ref[...] = jnp.dot(hn, cl1n_ref[...],
                             preferred_element_type=jnp.float32)


def _post(agg, h, cl2, bcl2, wlin, blin, cl1n):
    nb = _NP // 512
    full = lambda i: (0, 0)
    return pl.pallas_call(
        _post_body,
        grid=(nb,),
        in_specs=[
            pl.BlockSpec((512, H), lambda i: (i, 0)),
            pl.BlockSpec((512, H), lambda i: (i, 0)),
            pl.BlockSpec((F, H), full),
            pl.BlockSpec((H,), lambda i: (0,)),
            pl.BlockSpec((H, H), full),
            pl.BlockSpec((H,), lambda i: (0,)),
            pl.BlockSpec((H, F), full),
        ],
        out_specs=[
            pl.BlockSpec((512, H), lambda i: (i, 0)),
            pl.BlockSpec((512, F), lambda i: (i, 0)),
        ],
        out_shape=[
            jax.ShapeDtypeStruct((_NP, H), jnp.float32),
            jax.ShapeDtypeStruct((_NP, F), jnp.float32),
        ],
    )(agg, h, cl2, bcl2, wlin, blin, cl1n)


def _xs0_body(h_ref, w_ref, out_ref):
    out_ref[...] = jnp.dot(h_ref[...], w_ref[...],
                           preferred_element_type=jnp.float32)


def _xs0(h, w):
    return pl.pallas_call(
        _xs0_body,
        grid=(_NP // 512,),
        in_specs=[pl.BlockSpec((512, H), lambda i: (i, 0)),
                  pl.BlockSpec((H, F), lambda i: (0, 0))],
        out_specs=pl.BlockSpec((512, F), lambda i: (i, 0)),
        out_shape=jax.ShapeDtypeStruct((_NP, F), jnp.float32),
    )(h, w)


_RB = 512


def _readout_body(h_ref, b_ref, w1_ref, b1_ref, w2_ref, b2_ref, out_ref):
    i = pl.program_id(0)

    @pl.when(i == 0)
    def _():
        out_ref[...] = jnp.zeros_like(out_ref)

    hb = h_ref[...]
    x = _ssp(jnp.dot(hb, w1_ref[...], preferred_element_type=jnp.float32)
             + b1_ref[...])
    y = jnp.dot(x, w2_ref[...], preferred_element_type=jnp.float32) + b2_ref[...]
    bb = b_ref[0, 0, :]
    gids = jax.lax.broadcasted_iota(jnp.int32, (NG, _RB), 0)
    onehot = (bb[None, :] == gids).astype(jnp.float32)
    out_ref[...] += jnp.dot(onehot, y, preferred_element_type=jnp.float32)


def _readout(h, batch_padded, lin1_w, lin1_b, lin2_w, lin2_b):
    nb = _NP // _RB
    bp = batch_padded.reshape(nb, 1, _RB)
    out = pl.pallas_call(
        _readout_body,
        grid=(nb,),
        in_specs=[
            pl.BlockSpec((_RB, H), lambda i: (i, 0)),
            pl.BlockSpec((1, 1, _RB), lambda i: (i, 0, 0)),
            pl.BlockSpec((H, H // 2), lambda i: (0, 0)),
            pl.BlockSpec((H // 2,), lambda i: (0,)),
            pl.BlockSpec((H // 2, OUT), lambda i: (0, 0)),
            pl.BlockSpec((OUT,), lambda i: (0,)),
        ],
        out_specs=pl.BlockSpec((NG, OUT), lambda i: (0, 0)),
        out_shape=jax.ShapeDtypeStruct((NG, OUT), jnp.float32),
    )(h, bp, lin1_w, lin1_b, lin2_w, lin2_b)
    return out


def kernel(z, pos, batch, emb, iw_mlp1, ib_mlp1, iw_mlp2, ib_mlp2, iw_cl1,
           iw_cl2, ib_cl2, iw_lin, ib_lin, lin1_w, lin1_b, lin2_w, lin2_b,
           target_mean, target_std):
    posg = jax.lax.stop_gradient(pos)
    batch32 = batch.astype(jnp.int32)
    gids = jnp.arange(NG, dtype=jnp.int32)
    seg_start = jnp.searchsorted(batch32, gids, side="left").astype(jnp.int32)
    seg_end = jnp.searchsorted(batch32, gids, side="right").astype(jnp.int32)
    seg_len = seg_end - seg_start
    overflow = jnp.max(seg_len) > P

    src2, mask2 = jax.lax.cond(
        overflow,
        lambda: _build_graph_full(posg, batch32),
        lambda: _build_graph_windowed(posg, batch32, seg_start, seg_len),
    )
    src, mask = src2.reshape(-1), mask2.reshape(-1)

    dst = jnp.repeat(jnp.arange(N), K)
    diff = pos[dst] - pos[src]
    dist = jnp.sqrt(jnp.maximum(jnp.sum(diff * diff, axis=-1), 1e-12))
    C = 0.5 * (jnp.cos(dist * jnp.pi / CUT) + 1.0) * mask.astype(pos.dtype)

    pad_e = _EP - N * K
    dist_p = jnp.pad(dist, (0, pad_e), constant_values=1.0).reshape(_EP, 1)
    c_p = jnp.pad(C, (0, pad_e)).reshape(_EP, 1)
    src_p = jnp.pad(src, (0, pad_e)).astype(jnp.int32)
    h0 = jnp.pad(emb[z], ((0, _NP - N), (0, 0)))

    ws = jnp.clip(jnp.arange(_NW, dtype=jnp.int32) * _NDW - P,
                  0, _NP - _WIN)
    rel = jnp.clip(src_p - jnp.repeat(ws, _PER_W), 0, _WIN - 1)
    rel3 = rel.reshape(_NW, _NCH, _CH)

    def fast_layers():
        h = h0
        xs = _xs0(h, iw_cl1[0])
        zero_w = jnp.zeros((H, F), jnp.float32)
        for l in range(L):
            cl1n = iw_cl1[l + 1] if l + 1 < L else zero_w
            if _USE_SC:
                wf = _wf(dist_p, c_p, iw_mlp1[l], ib_mlp1[l],
                         iw_mlp2[l], ib_mlp2[l])
                agg = _sc_msg(xs, wf, rel3)
                h, xs = _post(agg, h, iw_cl2[l], ib_cl2[l],
                              iw_lin[l], ib_lin[l], cl1n)
            else:
                xg = xs[src_p]
                h, xs = _layer(dist_p, c_p, xg, h, iw_mlp1[l], ib_mlp1[l],
                               iw_mlp2[l], ib_mlp2[l], iw_cl2[l], ib_cl2[l],
                               iw_lin[l], ib_lin[l], cl1n)
        return h

    def slow_layers():
        h = h0[:N]
        offset = jnp.linspace(0.0, CUT, G)
        coeff = -0.5 / (offset[1] - offset[0]) ** 2
        edge_attr = jnp.exp(coeff * (dist[:, None] - offset[None, :]) ** 2)
        for l in range(L):
            Wf = (_ssp(edge_attr @ iw_mlp1[l] + ib_mlp1[l]) @ iw_mlp2[l]
                  + ib_mlp2[l]) * C[:, None]
            xs = h @ iw_cl1[l]
            msg = xs[src] * Wf
            agg = msg.reshape(N, K, H).sum(axis=1)
            conv = agg @ iw_cl2[l] + ib_cl2[l]
            h = h + (_ssp(conv) @ iw_lin[l] + ib_lin[l])
        return jnp.pad(h, ((0, _NP - N), (0, 0)))

    h_fast = fast_layers()
    h_slow = jax.lax.cond(
        overflow, slow_layers,
        lambda: jnp.zeros((_NP, H), jnp.float32))
    h = jnp.where(overflow, h_slow, h_fast)

    batch_p = jnp.pad(batch32, (0, _NP - N), constant_values=NG)
    out = _readout(h, batch_p, lin1_w, lin1_b, lin2_w, lin2_b)
    return out * target_std + target_mean
